# baseline (speedup 1.0000x reference)
import functools

import jax
import jax.numpy as jnp
from jax import lax
from jax.experimental import pallas as pl
from jax.experimental.pallas import tpu as pltpu
from jax.experimental.pallas import tpu_sc as plsc

_NUM_ENVS = 1024
_NUM_AGENTS = 32
_D_IN = 128
_D_GCN = 64
_E_PER = 512

_NW = 32
_EPW = _NUM_ENVS // _NW
_AA = _NUM_AGENTS * _NUM_AGENTS
_WPE = _E_PER // 2
_CPE = _AA // 2


def _tc_pack_edges(edge_index):
    B = 256

    def body(e_ref, out_ref):
        row = e_ref[:, 0, :]
        col = e_ref[:, 1, :]
        loc = col * _NUM_AGENTS + row
        out_ref[...] = loc[:, :_WPE] | (loc[:, _WPE:] << 16)

    return pl.pallas_call(
        body,
        grid=(_NUM_ENVS // B,),
        in_specs=[pl.BlockSpec((B, 2, _E_PER), lambda i: (i, 0, 0))],
        out_specs=pl.BlockSpec((B, _WPE), lambda i: (i, 0)),
        out_shape=jax.ShapeDtypeStruct((_NUM_ENVS, _WPE), jnp.int32),
    )(edge_index)


def _sc_edge_counts(ew_flat):
    mesh = plsc.VectorSubcoreMesh(core_axis_name="c", subcore_axis_name="s")

    @functools.partial(
        pl.kernel,
        mesh=mesh,
        compiler_params=pltpu.CompilerParams(needs_layout_passes=False),
        out_type=jax.ShapeDtypeStruct((_NUM_ENVS * _CPE,), jnp.int32),
        scratch_types=[
            pltpu.VMEM((_EPW * _WPE,), jnp.int32),
            pltpu.VMEM((_EPW * _AA,), jnp.int32),
            pltpu.VMEM((_EPW * _CPE,), jnp.int32),
        ],
    )
    def k(ew_hbm, out_hbm, ew_v, acc_v, pack_v):
        wid = lax.axis_index("s") * 2 + lax.axis_index("c")
        base = wid * _EPW
        pltpu.sync_copy(ew_hbm.at[pl.ds(base * _WPE, _EPW * _WPE)], ew_v)

        zeros = jnp.zeros((16,), jnp.int32)
        ones = jnp.ones((16,), jnp.int32)
        iota = lax.iota(jnp.int32, 16)

        def zero_chunk(c, carry):
            acc_v[pl.ds(c * 16, 16)] = zeros
            return carry

        lax.fori_loop(0, _EPW * _AA // 16, zero_chunk, 0, unroll=8)

        def do_env(e, carry):
            ebase = e * _WPE
            abase = e * _AA

            def do_chunk(c, carry2):
                w16 = ew_v[pl.ds(ebase + c * 16, 16)]
                lo = (w16 & 0xFFFF) + abase
                hi = lax.shift_right_logical(w16, 16) + abase
                plsc.addupdate_scatter(acc_v, [lo], ones)
                plsc.addupdate_scatter(acc_v, [hi], ones)
                return carry2

            lax.fori_loop(0, _WPE // 16, do_chunk, 0, unroll=4)
            plsc.addupdate_scatter(acc_v, [iota * 33 + abase], ones)
            plsc.addupdate_scatter(acc_v, [(iota + 16) * 33 + abase], ones)
            return carry

        lax.fori_loop(0, _EPW, do_env, 0)

        def pack_env(e, carry):
            abase = e * _AA
            pbase = e * _CPE

            def pack_chunk(c, carry2):
                a = acc_v[pl.ds(abase + c * 16, 16)]
                b = acc_v[pl.ds(abase + _CPE + c * 16, 16)]
                pack_v[pl.ds(pbase + c * 16, 16)] = a | (b << 16)
                return carry2

            return lax.fori_loop(0, _CPE // 16, pack_chunk, carry, unroll=4)

        lax.fori_loop(0, _EPW, pack_env, 0)
        pltpu.sync_copy(pack_v, out_hbm.at[pl.ds(base * _CPE, _EPW * _CPE)])

    return k(ew_flat)


_E_B = 128


def _tc_input_proj(x2, W_xh, b_h):
    B = 8192

    def body(x_ref, w_ref, b_ref, o_ref):
        o_ref[...] = jnp.dot(x_ref[...], w_ref[...],
                             preferred_element_type=jnp.float32) + b_ref[...]

    return pl.pallas_call(
        body,
        grid=(_NUM_ENVS * _NUM_AGENTS // B,),
        in_specs=[
            pl.BlockSpec((B, _D_IN), lambda i: (i, 0)),
            pl.BlockSpec((_D_IN, _D_GCN), lambda i: (0, 0)),
            pl.BlockSpec((1, _D_GCN), lambda i: (0, 0)),
        ],
        out_specs=pl.BlockSpec((B, _D_GCN), lambda i: (i, 0)),
        out_shape=jax.ShapeDtypeStruct((_NUM_ENVS * _NUM_AGENTS, _D_GCN), jnp.float32),
    )(x2, W_xh, b_h)


def _tc_forward(hin, Cw, W_ir, W_iz, W_in, c_r, c_z, c_n, b_hn, Wl, b_l):
    grid = _NUM_ENVS // _E_B
    R = _E_B * _NUM_AGENTS

    def body(h_ref, c_ref, wir_ref, wiz_ref, win_ref,
             cr_ref, cz_ref, cn_ref, bhn_ref, wl_ref, bl_ref,
             val_ref, hid_ref):
        h = h_ref[...]
        w = c_ref[...]
        clo = (w & 0xFFFF).astype(jnp.float32)
        chi = lax.shift_right_logical(w, 16).astype(jnp.float32)
        Cb = jnp.concatenate([clo, chi], axis=1).reshape(_E_B, _NUM_AGENTS, _NUM_AGENTS)
        deg = jnp.sum(Cb, axis=2)
        dis = lax.rsqrt(deg)
        h3 = h.reshape(_E_B, _NUM_AGENTS, _D_GCN) * dis[:, :, None]
        m = lax.dot_general(Cb, h3, (((2,), (1,)), ((0,), (0,))),
                            preferred_element_type=jnp.float32)
        m = m * dis[:, :, None]
        m2 = m.reshape(R, _D_GCN)
        r = jax.nn.sigmoid(jnp.dot(m2, wir_ref[...], preferred_element_type=jnp.float32) + cr_ref[...])
        z = jax.nn.sigmoid(jnp.dot(m2, wiz_ref[...], preferred_element_type=jnp.float32) + cz_ref[...])
        n = jnp.tanh(jnp.dot(m2, win_ref[...], preferred_element_type=jnp.float32)
                     + cn_ref[...] + r * bhn_ref[...])
        hn = (1.0 - z) * n
        hn3 = hn.reshape(_E_B, _NUM_AGENTS, _D_GCN)
        hnT = jnp.concatenate(
            [jnp.transpose(hn3[:, a, :]) for a in range(_NUM_AGENTS)], axis=0)
        hid_ref[...] = hnT
        val_ref[...] = jnp.dot(wl_ref[...], hnT,
                               preferred_element_type=jnp.float32) + bl_ref[...]

    full = lambda shape: pl.BlockSpec(shape, lambda i: (0,) * len(shape))
    return pl.pallas_call(
        body,
        grid=(grid,),
        in_specs=[
            pl.BlockSpec((R, _D_GCN), lambda i: (i, 0)),
            pl.BlockSpec((_E_B, _CPE), lambda i: (i, 0)),
            full((_D_GCN, _D_GCN)),
            full((_D_GCN, _D_GCN)),
            full((_D_GCN, _D_GCN)),
            full((1, _D_GCN)),
            full((1, _D_GCN)),
            full((1, _D_GCN)),
            full((1, _D_GCN)),
            full((1, _NUM_AGENTS * _D_GCN)),
            full((1, 1)),
        ],
        out_specs=[
            pl.BlockSpec((1, _E_B), lambda i: (0, i)),
            pl.BlockSpec((_NUM_AGENTS * _D_GCN, _E_B), lambda i: (0, i)),
        ],
        out_shape=[
            jax.ShapeDtypeStruct((1, _NUM_ENVS), jnp.float32),
            jax.ShapeDtypeStruct((_NUM_AGENTS * _D_GCN, _NUM_ENVS), jnp.float32),
        ],
    )(hin, Cw, W_ir, W_iz, W_in, c_r, c_z, c_n, b_hn, Wl, b_l)


def kernel(x, edge_index, W_pre, b_pre, W_gcn, b_gcn, W_ih, W_hh, b_ih, b_hh, W_lin, b_lin):
    ew = _tc_pack_edges(edge_index.astype(jnp.int32))
    Cp = _sc_edge_counts(ew.reshape(_NUM_ENVS * _WPE))
    Cw = Cp.reshape(_NUM_ENVS, _CPE)

    x2 = x.reshape(_NUM_ENVS * _NUM_AGENTS, _D_IN)
    W_xh = W_pre.T @ W_gcn.T
    b_h = (b_pre @ W_gcn.T)[None]
    W_ir = W_ih[:_D_GCN].T
    W_iz = W_ih[_D_GCN:2 * _D_GCN].T
    W_in = W_ih[2 * _D_GCN:].T
    c_r = (b_ih[:_D_GCN] + b_hh[:_D_GCN] + b_gcn @ W_ih[:_D_GCN].T)[None]
    c_z = (b_ih[_D_GCN:2 * _D_GCN] + b_hh[_D_GCN:2 * _D_GCN]
           + b_gcn @ W_ih[_D_GCN:2 * _D_GCN].T)[None]
    c_n = (b_ih[2 * _D_GCN:] + b_gcn @ W_ih[2 * _D_GCN:].T)[None]
    b_hn = b_hh[2 * _D_GCN:][None]
    b_l = b_lin.reshape(1, 1)

    hin = _tc_input_proj(x2, W_xh, b_h)
    valT, hidT = _tc_forward(hin, Cw, W_ir, W_iz, W_in,
                             c_r, c_z, c_n, b_hn, W_lin, b_l)
    value = valT.reshape(_NUM_ENVS, 1)
    next_hidden = jnp.transpose(
        hidT.reshape(_NUM_AGENTS, _D_GCN, _NUM_ENVS), (2, 0, 1))
    return (value, next_hidden)


# speedup vs baseline: 1.0773x; 1.0773x over previous
import functools

import jax
import jax.numpy as jnp
from jax import lax
from jax.experimental import pallas as pl
from jax.experimental.pallas import tpu as pltpu
from jax.experimental.pallas import tpu_sc as plsc

_NUM_ENVS = 1024
_NUM_AGENTS = 32
_D_IN = 128
_D_GCN = 64
_E_PER = 512

_NW = 32
_EPW = _NUM_ENVS // _NW
_AA = _NUM_AGENTS * _NUM_AGENTS
_WPE = _E_PER // 2
_CPE = _AA // 2


def _tc_pack_edges(edge_index):
    B = 256

    def body(e_ref, out_ref):
        row = e_ref[:, 0, :]
        col = e_ref[:, 1, :]
        loc = col * _NUM_AGENTS + row
        out_ref[...] = loc[:, :_WPE] | (loc[:, _WPE:] << 16)

    return pl.pallas_call(
        body,
        grid=(_NUM_ENVS // B,),
        in_specs=[pl.BlockSpec((B, 2, _E_PER), lambda i: (i, 0, 0))],
        out_specs=pl.BlockSpec((B, _WPE), lambda i: (i, 0)),
        out_shape=jax.ShapeDtypeStruct((_NUM_ENVS, _WPE), jnp.int32),
    )(edge_index)


def _sc_edge_counts(ew_flat):
    mesh = plsc.VectorSubcoreMesh(core_axis_name="c", subcore_axis_name="s")

    @functools.partial(
        pl.kernel,
        mesh=mesh,
        compiler_params=pltpu.CompilerParams(needs_layout_passes=False),
        out_type=jax.ShapeDtypeStruct((_NUM_ENVS * _CPE,), jnp.int32),
        scratch_types=[
            pltpu.VMEM((_EPW * _WPE,), jnp.int32),
            pltpu.VMEM((_EPW * _AA,), jnp.int32),
            pltpu.VMEM((_EPW * _CPE,), jnp.int32),
        ],
    )
    def k(ew_hbm, out_hbm, ew_v, acc_v, pack_v):
        wid = lax.axis_index("s") * 2 + lax.axis_index("c")
        base = wid * _EPW
        pltpu.sync_copy(ew_hbm.at[pl.ds(base * _WPE, _EPW * _WPE)], ew_v)

        zeros = jnp.zeros((16,), jnp.int32)
        ones = jnp.ones((16,), jnp.int32)
        iota = lax.iota(jnp.int32, 16)

        def zero_chunk(c, carry):
            acc_v[pl.ds(c * 16, 16)] = zeros
            return carry

        lax.fori_loop(0, _EPW * _AA // 16, zero_chunk, 0, unroll=8)

        def do_env(e, carry):
            ebase = e * _WPE
            abase = e * _AA

            def do_chunk(c, carry2):
                w16 = ew_v[pl.ds(ebase + c * 16, 16)]
                lo = (w16 & 0xFFFF) + abase
                hi = lax.shift_right_logical(w16, 16) + abase
                plsc.addupdate_scatter(acc_v, [lo], ones)
                plsc.addupdate_scatter(acc_v, [hi], ones)
                return carry2

            lax.fori_loop(0, _WPE // 16, do_chunk, 0, unroll=4)
            plsc.addupdate_scatter(acc_v, [iota * 33 + abase], ones)
            plsc.addupdate_scatter(acc_v, [(iota + 16) * 33 + abase], ones)
            return carry

        lax.fori_loop(0, _EPW, do_env, 0)

        def pack_env(e, carry):
            abase = e * _AA
            pbase = e * _CPE

            def pack_chunk(c, carry2):
                a = acc_v[pl.ds(abase + c * 16, 16)]
                b = acc_v[pl.ds(abase + _CPE + c * 16, 16)]
                pack_v[pl.ds(pbase + c * 16, 16)] = a | (b << 16)
                return carry2

            return lax.fori_loop(0, _CPE // 16, pack_chunk, carry, unroll=4)

        lax.fori_loop(0, _EPW, pack_env, 0)
        pltpu.sync_copy(pack_v, out_hbm.at[pl.ds(base * _CPE, _EPW * _CPE)])

    return k(ew_flat)


_E_B = 128


def _tc_forward(x2, Cw, W_xh, b_h, W_ir, W_iz, W_in, c_r, c_z, c_n, b_hn, Wl, b_l):
    grid = _NUM_ENVS // _E_B
    R = _E_B * _NUM_AGENTS

    def body(x_ref, c_ref, wxh_ref, bh_ref, wir_ref, wiz_ref, win_ref,
             cr_ref, cz_ref, cn_ref, bhn_ref, wl_ref, bl_ref,
             val_ref, hid_ref):
        h = jnp.dot(x_ref[...], wxh_ref[...],
                    preferred_element_type=jnp.float32) + bh_ref[...]
        w = c_ref[...]
        clo = (w & 0xFFFF).astype(jnp.float32)
        chi = lax.shift_right_logical(w, 16).astype(jnp.float32)
        Cb = jnp.concatenate([clo, chi], axis=1).reshape(_E_B, _NUM_AGENTS, _NUM_AGENTS)
        deg = jnp.sum(Cb, axis=2)
        dis = lax.rsqrt(deg)
        h3 = h.reshape(_E_B, _NUM_AGENTS, _D_GCN) * dis[:, :, None]
        m = lax.dot_general(Cb, h3, (((2,), (1,)), ((0,), (0,))),
                            preferred_element_type=jnp.float32)
        m = m * dis[:, :, None]
        m2 = m.reshape(R, _D_GCN)
        r = jax.nn.sigmoid(jnp.dot(m2, wir_ref[...], preferred_element_type=jnp.float32) + cr_ref[...])
        z = jax.nn.sigmoid(jnp.dot(m2, wiz_ref[...], preferred_element_type=jnp.float32) + cz_ref[...])
        n = jnp.tanh(jnp.dot(m2, win_ref[...], preferred_element_type=jnp.float32)
                     + cn_ref[...] + r * bhn_ref[...])
        hn = (1.0 - z) * n
        hn3 = hn.reshape(_E_B, _NUM_AGENTS, _D_GCN)
        hnT = jnp.concatenate(
            [jnp.transpose(hn3[:, a, :]) for a in range(_NUM_AGENTS)], axis=0)
        hid_ref[...] = hnT
        val_ref[...] = jnp.dot(wl_ref[...], hnT,
                               preferred_element_type=jnp.float32) + bl_ref[...]

    full = lambda shape: pl.BlockSpec(shape, lambda i: (0,) * len(shape))
    return pl.pallas_call(
        body,
        grid=(grid,),
        in_specs=[
            pl.BlockSpec((R, _D_IN), lambda i: (i, 0)),
            pl.BlockSpec((_E_B, _CPE), lambda i: (i, 0)),
            full((_D_IN, _D_GCN)),
            full((1, _D_GCN)),
            full((_D_GCN, _D_GCN)),
            full((_D_GCN, _D_GCN)),
            full((_D_GCN, _D_GCN)),
            full((1, _D_GCN)),
            full((1, _D_GCN)),
            full((1, _D_GCN)),
            full((1, _D_GCN)),
            full((1, _NUM_AGENTS * _D_GCN)),
            full((1, 1)),
        ],
        out_specs=[
            pl.BlockSpec((1, _E_B), lambda i: (0, i)),
            pl.BlockSpec((_NUM_AGENTS * _D_GCN, _E_B), lambda i: (0, i)),
        ],
        out_shape=[
            jax.ShapeDtypeStruct((1, _NUM_ENVS), jnp.float32),
            jax.ShapeDtypeStruct((_NUM_AGENTS * _D_GCN, _NUM_ENVS), jnp.float32),
        ],
    )(x2, Cw, W_xh, b_h, W_ir, W_iz, W_in, c_r, c_z, c_n, b_hn, Wl, b_l)


def kernel(x, edge_index, W_pre, b_pre, W_gcn, b_gcn, W_ih, W_hh, b_ih, b_hh, W_lin, b_lin):
    ew = _tc_pack_edges(edge_index.astype(jnp.int32))
    Cp = _sc_edge_counts(ew.reshape(_NUM_ENVS * _WPE))
    Cw = Cp.reshape(_NUM_ENVS, _CPE)

    x2 = x.reshape(_NUM_ENVS * _NUM_AGENTS, _D_IN)
    W_xh = W_pre.T @ W_gcn.T
    b_h = (b_pre @ W_gcn.T)[None]
    W_ir = W_ih[:_D_GCN].T
    W_iz = W_ih[_D_GCN:2 * _D_GCN].T
    W_in = W_ih[2 * _D_GCN:].T
    c_r = (b_ih[:_D_GCN] + b_hh[:_D_GCN] + b_gcn @ W_ih[:_D_GCN].T)[None]
    c_z = (b_ih[_D_GCN:2 * _D_GCN] + b_hh[_D_GCN:2 * _D_GCN]
           + b_gcn @ W_ih[_D_GCN:2 * _D_GCN].T)[None]
    c_n = (b_ih[2 * _D_GCN:] + b_gcn @ W_ih[2 * _D_GCN:].T)[None]
    b_hn = b_hh[2 * _D_GCN:][None]
    b_l = b_lin.reshape(1, 1)

    valT, hidT = _tc_forward(x2, Cw, W_xh, b_h, W_ir, W_iz, W_in,
                             c_r, c_z, c_n, b_hn, W_lin, b_l)
    value = valT.reshape(_NUM_ENVS, 1)
    next_hidden = jnp.transpose(
        hidT.reshape(_NUM_AGENTS, _D_GCN, _NUM_ENVS), (2, 0, 1))
    return (value, next_hidden)
